# Initial kernel scaffold; baseline (speedup 1.0000x reference)
import jax
import jax.numpy as jnp
from jax.experimental import pallas as pl


def kernel(x, edge_index, pos, params, W_fc, b_fc):
    raise NotImplementedError("write your pallas kernel here")


# speedup vs baseline: 22.1630x; 22.1630x over previous
import functools

import jax
import jax.numpy as jnp
from jax import lax
from jax.experimental import pallas as pl
from jax.experimental.pallas import tpu as pltpu
from jax.experimental.pallas import tpu_sc as plsc

N = 10000
F = 128
D = 3
NPAD = 10240
NW = 32
KJ = 79
EPW = KJ * 128
EPAD = NW * EPW
RPT = NPAD // 16
BN = 1024
GRID = NPAD // BN

_HI = jax.lax.Precision.DEFAULT


def _sc_segsum(h, src_b, dst_b):
    mesh = plsc.VectorSubcoreMesh(core_axis_name="c", subcore_axis_name="s")

    @functools.partial(
        pl.kernel,
        out_type=jax.ShapeDtypeStruct((2, NPAD, F), jnp.float32),
        mesh=mesh,
        scratch_types=[
            pltpu.VMEM((KJ, 128), jnp.int32),
            pltpu.VMEM((KJ, 128), jnp.int32),
            pltpu.VMEM((128, F), jnp.float32),
            pltpu.VMEM_SHARED((NPAD, F), jnp.float32),
            pltpu.SemaphoreType.DMA,
        ],
    )
    def k(h_hbm, src_hbm, dst_hbm, out_hbm, sidx, didx, rows, acc, sem):
        c = lax.axis_index("c")
        s = lax.axis_index("s")
        wid = c * 16 + s

        zero = jnp.zeros((16,), jnp.float32)

        def zrow(i, _):
            def zcol(j, _):
                rows[i, pl.ds(j * 16, 16)] = zero
                return _
            return lax.fori_loop(0, F // 16, zcol, _)
        lax.fori_loop(0, 128, zrow, None)

        def zchunk(m, _):
            pltpu.sync_copy(rows, acc.at[pl.ds(s * RPT + m * 128, 128)])
            return _
        lax.fori_loop(0, RPT // 128, zchunk, None)
        plsc.subcore_barrier()

        pltpu.sync_copy(src_hbm.at[wid], sidx)
        pltpu.sync_copy(dst_hbm.at[wid], didx)

        def body(j, _):
            pltpu.async_copy(h_hbm.at[sidx.at[j]], rows, sem).wait()
            pltpu.sync_copy(rows, acc.at[didx.at[j]], add=True)
            return _
        lax.fori_loop(0, KJ, body, None)

        plsc.subcore_barrier()

        def wb(m, _):
            r0 = s * RPT + m * 128
            pltpu.sync_copy(acc.at[pl.ds(r0, 128)], out_hbm.at[c].at[pl.ds(r0, 128)])
            return _
        lax.fori_loop(0, RPT // 128, wb, None)

    return k(h, src_b, dst_b)


def _mm_body(x_ref, w_ref, b_ref, o_ref):
    o_ref[...] = (
        jnp.dot(x_ref[...], w_ref[...], precision=_HI,
                preferred_element_type=jnp.float32) + b_ref[...])


def _mm(x, w, b):
    return pl.pallas_call(
        _mm_body,
        grid=(GRID,),
        in_specs=[
            pl.BlockSpec((BN, F), lambda i: (i, 0)),
            pl.BlockSpec((F, F), lambda i: (0, 0)),
            pl.BlockSpec((1, F), lambda i: (0, 0)),
        ],
        out_specs=pl.BlockSpec((BN, F), lambda i: (i, 0)),
        out_shape=jax.ShapeDtypeStruct((NPAD, F), jnp.float32),
    )(x, w, b)


def _house_q(m):
    a0, b0 = m[0], m[1]
    a1, b1 = m[3], m[4]
    a2, b2 = m[6], m[7]
    one = jnp.ones_like(a0)
    zero = jnp.zeros_like(a0)

    sig1 = a1 * a1 + a2 * a2
    mu1 = jnp.sqrt(a0 * a0 + sig1)
    beta1 = jnp.where(a0 >= 0, -mu1, mu1)
    nz1 = sig1 > 0
    d1 = jnp.where(nz1, a0 - beta1, one)
    tau1 = jnp.where(nz1, (beta1 - a0) / beta1, zero)
    u1 = a1 / d1
    u2 = a2 / d1

    dot1 = b0 + u1 * b1 + u2 * b2
    y1 = b1 - tau1 * u1 * dot1
    y2 = b2 - tau1 * u2 * dot1
    sig2 = y2 * y2
    mu2 = jnp.sqrt(y1 * y1 + sig2)
    beta2 = jnp.where(y1 >= 0, -mu2, mu2)
    nz2 = sig2 > 0
    d2 = jnp.where(nz2, y1 - beta2, one)
    tau2 = jnp.where(nz2, (beta2 - y1) / beta2, zero)
    w2 = y2 / d2

    h11 = 1.0 - tau2
    h12 = -tau2 * w2
    h22 = 1.0 - tau2 * w2 * w2

    def apply_h1(x0, x1, x2):
        d = x0 + u1 * x1 + u2 * x2
        return x0 - tau1 * d, x1 - tau1 * u1 * d, x2 - tau1 * u2 * d

    q00, q10, q20 = apply_h1(one, zero, zero)
    q01, q11, q21 = apply_h1(zero, h11, h12)
    q02, q12, q22 = apply_h1(zero, h12, h22)
    return [q00, q01, q02, q10, q11, q12, q20, q21, q22]


def _layer_body(p_ref, rp_ref, t_ref, pos_ref, wn_ref, bn_ref, wc_ref, bc_ref,
                h_ref, rn_ref, tn_ref, posn_ref):
    agg = p_ref[0] + p_ref[1]
    h_ref[...] = (
        jnp.dot(agg, wn_ref[...], precision=_HI,
                preferred_element_type=jnp.float32) + bn_ref[...])
    upd = (jnp.dot(agg, wc_ref[...], precision=_HI,
                   preferred_element_type=jnp.float32) + bc_ref[...])
    coord = upd[:, 0:1]
    rotm = [upd[:, 1 + k:2 + k] for k in range(9)]
    q = _house_q(rotm)
    for r in range(3):
        for c in range(3):
            acc = q[3 * r + 0] * rp_ref[:, 0 + c:1 + c]
            acc = acc + q[3 * r + 1] * rp_ref[:, 3 + c:4 + c]
            acc = acc + q[3 * r + 2] * rp_ref[:, 6 + c:7 + c]
            rn_ref[:, 3 * r + c:3 * r + c + 1] = acc
    tn_ref[...] = t_ref[...] + upd[:, 10:13]
    posn_ref[...] = pos_ref[...] + coord


def _layer(p, r_prev, t_prev, pos_prev, w_next, b_next, w_cat, b_cat):
    return pl.pallas_call(
        _layer_body,
        grid=(GRID,),
        in_specs=[
            pl.BlockSpec((2, BN, F), lambda i: (0, i, 0)),
            pl.BlockSpec((BN, 9), lambda i: (i, 0)),
            pl.BlockSpec((BN, D), lambda i: (i, 0)),
            pl.BlockSpec((BN, D), lambda i: (i, 0)),
            pl.BlockSpec((F, F), lambda i: (0, 0)),
            pl.BlockSpec((1, F), lambda i: (0, 0)),
            pl.BlockSpec((F, 16), lambda i: (0, 0)),
            pl.BlockSpec((1, 16), lambda i: (0, 0)),
        ],
        out_specs=[
            pl.BlockSpec((BN, F), lambda i: (i, 0)),
            pl.BlockSpec((BN, 9), lambda i: (i, 0)),
            pl.BlockSpec((BN, D), lambda i: (i, 0)),
            pl.BlockSpec((BN, D), lambda i: (i, 0)),
        ],
        out_shape=[
            jax.ShapeDtypeStruct((NPAD, F), jnp.float32),
            jax.ShapeDtypeStruct((NPAD, 9), jnp.float32),
            jax.ShapeDtypeStruct((NPAD, D), jnp.float32),
            jax.ShapeDtypeStruct((NPAD, D), jnp.float32),
        ],
    )(p, r_prev, t_prev, pos_prev, w_next, b_next, w_cat, b_cat)


def kernel(x, edge_index, pos, params, W_fc, b_fc):
    src = edge_index[0].astype(jnp.int32)
    dst = edge_index[1].astype(jnp.int32)
    epad = EPAD - src.shape[0]
    src_b = jnp.concatenate(
        [src, (jnp.arange(epad, dtype=jnp.int32) * 131) % N]).reshape(NW, KJ, 128)
    dst_b = jnp.concatenate(
        [dst, jnp.full((epad,), NPAD - 1, jnp.int32)]).reshape(NW, KJ, 128)

    xp = jnp.zeros((NPAD, F), jnp.float32).at[:N].set(x)
    posp = jnp.zeros((NPAD, D), jnp.float32).at[:N].set(pos)
    r_flat = jnp.broadcast_to(
        jnp.eye(D, dtype=jnp.float32).reshape(1, 9), (NPAD, 9))
    t = jnp.zeros((NPAD, D), jnp.float32)

    h = _mm(xp, params[0]['W_lin'], params[0]['b_lin'].reshape(1, F))
    for l in range(4):
        p = _sc_segsum(h, src_b, dst_b)
        if l < 3:
            w_next = params[l + 1]['W_lin']
            b_next = params[l + 1]['b_lin'].reshape(1, F)
        else:
            w_next = W_fc
            b_next = b_fc.reshape(1, F)
        pr = params[l]
        w_cat = jnp.concatenate(
            [pr['W_coord'], pr['W_rot'], pr['W_trans'],
             jnp.zeros((F, 3), jnp.float32)], axis=1)
        b_cat = jnp.concatenate(
            [pr['b_coord'], pr['b_rot'], pr['b_trans'],
             jnp.zeros((3,), jnp.float32)]).reshape(1, 16)
        h, r_flat, t, posp = _layer(
            p, r_flat, t, posp, w_next, b_next, w_cat, b_cat)

    return (h[:N], posp[:N], r_flat[:N].reshape(N, D, D), t[:N])
